# baseline (speedup 1.0000x reference)
import functools

import jax
import jax.numpy as jnp
from jax import lax
from jax.experimental import pallas as pl
from jax.experimental.pallas import tpu as pltpu
from jax.experimental.pallas import tpu_sc as plsc

_EPS = 1e-05
_BT = 256
_BT_LOG2 = 8
_L = 16
_GR = 32


def _router_body(x_ref, tc_ref, tpw_ref, tpb_ref, gw_ref, idx_ref, w_ref, *, n_exp):
    x_r = x_ref[...] + jax.lax.dot_general(
        tc_ref[...], tpw_ref[...], (((1,), (1,)), ((), ())),
        preferred_element_type=jnp.float32) + tpb_ref[...]
    logits = jax.lax.dot_general(x_r, gw_ref[...], (((1,), (1,)), ((), ())),
                                 preferred_element_type=jnp.float32)
    m = jnp.max(logits, axis=1, keepdims=True)
    p = jnp.exp(logits - m)
    probs = p / jnp.sum(p, axis=1, keepdims=True)
    it = jax.lax.broadcasted_iota(jnp.int32, probs.shape, 1)
    m0 = jnp.max(probs, axis=1, keepdims=True)
    i0 = jnp.min(jnp.where(probs == m0, it, n_exp), axis=1, keepdims=True)
    pm = jnp.where(it == i0, -1.0, probs)
    m1 = jnp.max(pm, axis=1, keepdims=True)
    i1 = jnp.min(jnp.where(pm == m1, it, n_exp), axis=1, keepdims=True)
    denom = m0 + m1 + _EPS
    w0 = m0 / denom
    w1 = m1 / denom
    it2 = jax.lax.broadcasted_iota(jnp.int32, idx_ref.shape, 1)
    idx_ref[...] = jnp.where(it2 == 0, i0, i1)
    w_ref[...] = jnp.where(it2 == 0, w0, w1)


def _route_body(ebuf_hbm, gidx_hbm, p0_hbm, p1_hbm, bexp_hbm,
                ebuf, posbuf, tokbuf, stage, p0s, p1s, bexpv, sem,
                *, n_pairs, n_exp, n_pad, nb_routed, nb_pad):
    wid = lax.axis_index("s") * 2 + lax.axis_index("c")

    @pl.when(wid == 0)
    def _work():
        lanes = lax.iota(jnp.int32, _L)
        nv = n_pairs // _L

        pltpu.sync_copy(ebuf_hbm, ebuf)

        def hist_step(i, tot):
            v = ebuf[pl.ds(i * _L, _L)]
            for e in range(n_exp):
                cnt = jnp.sum(jnp.where(v == e, 1, 0))
                tot = jnp.where(lanes == e, tot + cnt, tot)
            return tot
        tot = lax.fori_loop(0, nv, hist_step, jnp.zeros((_L,), jnp.int32))

        inc = jax.lax.shift_left(
            jax.lax.shift_right_logical(tot + (_BT - 1), _BT_LOG2), _BT_LOG2)
        pstart = plsc.cumsum(inc) - inc

        pf = [jnp.sum(jnp.where(lanes == f, pstart, 0)) for f in range(n_exp)]
        nbv = nb_pad // _L
        for i in range(nbv):
            bv = lanes + i * _L
            start = bv * _BT
            cnt = jnp.zeros((_L,), jnp.int32)
            for f in range(n_exp):
                cnt = cnt + jnp.where(start >= pf[f], 1, 0)
            ev = jnp.minimum(jnp.maximum(cnt - 1, 0), n_exp - 1)
            bexpv[pl.ds(i * _L, _L)] = ev
        pltpu.sync_copy(bexpv, bexp_hbm)

        def zfill(i, c):
            stage[pl.ds(i * _L, _L)] = jnp.zeros((_L,), jnp.int32)
            return c
        lax.fori_loop(0, n_pairs // _L, zfill, 0)
        pltpu.sync_copy(stage, gidx_hbm.at[pl.ds(0, n_pairs)])
        rest = n_pad - n_pairs
        pltpu.sync_copy(stage.at[pl.ds(0, rest)],
                        gidx_hbm.at[pl.ds(n_pairs, rest)])

        def pos_step(i, cur):
            v = ebuf[pl.ds(i * _L, _L)]
            pos = jnp.zeros((_L,), jnp.int32)
            for e in range(n_exp):
                msk = v == e
                mi = jnp.where(msk, 1, 0)
                pre = plsc.cumsum(mi) - mi
                cnt = jnp.sum(mi)
                base = jnp.sum(jnp.where(lanes == e, cur, 0))
                pos = jnp.where(msk, base + pre, pos)
                cur = jnp.where(lanes == e, cur + cnt, cur)
            row = jax.lax.shift_right_logical(i, 3)
            col = jax.lax.shift_left(jnp.bitwise_and(i, 7), 4)
            posbuf[row, pl.ds(col, _L)] = pos
            tokbuf[row, pl.ds(col, _L)] = jax.lax.shift_right_logical(
                i * _L + lanes, 1)
            return cur
        lax.fori_loop(0, nv, pos_step, pstart)

        nrow = n_pairs // 128

        def sc_out(r8, c):
            for r2 in range(8):
                r = r8 * 8 + r2
                pltpu.async_copy(tokbuf.at[r], gidx_hbm.at[posbuf.at[r]], sem)
            for r2 in range(8):
                r = r8 * 8 + r2
                pltpu.make_async_copy(
                    tokbuf.at[r], gidx_hbm.at[posbuf.at[r]], sem).wait()
            return c
        lax.fori_loop(0, nrow // 8, sc_out, 0)

        def dei(g, c):
            slot = jax.lax.shift_left(g * _L + lanes, 1)
            r0 = jax.lax.shift_right_logical(slot, 7)
            c0 = jnp.bitwise_and(slot, 127)
            p0s[pl.ds(g * _L, _L)] = plsc.load_gather(posbuf, [r0, c0])
            slot1 = slot + 1
            r1 = jax.lax.shift_right_logical(slot1, 7)
            c1 = jnp.bitwise_and(slot1, 127)
            p1s[pl.ds(g * _L, _L)] = plsc.load_gather(posbuf, [r1, c1])
            return c
        lax.fori_loop(0, (n_pairs // 2) // _L, dei, 0)
        pltpu.sync_copy(p0s, p0_hbm)
        pltpu.sync_copy(p1s, p1_hbm)


def _gather_body(x_hbm, gidx_hbm, xs_hbm, gxbuf, rows0, rows1, semA, semB,
                 *, n_groups):
    wid = lax.axis_index("s") * 2 + lax.axis_index("c")
    per_w = n_groups // 32
    g0 = wid * per_w
    pltpu.sync_copy(gidx_hbm.at[pl.ds(g0 * _GR, per_w * _GR)], gxbuf)

    def idx(g):
        return gxbuf.at[pl.ds(g * _GR, _GR)]

    pltpu.async_copy(x_hbm.at[idx(0)], rows0, semA)

    def step(k, c):
        ge = 2 * k
        go = 2 * k + 1
        pltpu.async_copy(x_hbm.at[idx(go)], rows1, semB)
        pltpu.make_async_copy(x_hbm.at[idx(ge)], rows0, semA).wait()
        pltpu.sync_copy(rows0, xs_hbm.at[pl.ds((g0 + ge) * _GR, _GR)])

        @pl.when(k < per_w // 2 - 1)
        def _prefetch():
            pltpu.async_copy(x_hbm.at[idx(ge + 2)], rows0, semA)

        pltpu.make_async_copy(x_hbm.at[idx(go)], rows1, semB).wait()
        pltpu.sync_copy(rows1, xs_hbm.at[pl.ds((g0 + go) * _GR, _GR)])
        return c
    lax.fori_loop(0, per_w // 2, step, 0)


def _moe_body(be_ref, xs_ref, wg_ref, wu_ref, wd_ref, out_ref, *, n_i, bi):
    xb = xs_ref[...]
    y = jnp.zeros(out_ref.shape, jnp.float32)
    for k in range(n_i):
        sl = pl.ds(k * bi, bi)
        a = jax.lax.dot_general(xb, wg_ref[0, sl, :], (((1,), (1,)), ((), ())),
                                preferred_element_type=jnp.float32)
        b = jax.lax.dot_general(xb, wu_ref[0, sl, :], (((1,), (1,)), ((), ())),
                                preferred_element_type=jnp.float32)
        hh = (a * jax.lax.logistic(a) * b).astype(jnp.bfloat16)
        y = y + jax.lax.dot_general(hh, wd_ref[0, :, sl],
                                    (((1,), (1,)), ((), ())),
                                    preferred_element_type=jnp.float32)
    out_ref[...] = y


def _shared_body(x_ref, wg_ref, wu_ref, wd_ref, out_ref, *, n_i, bi):
    xb = x_ref[...].astype(jnp.bfloat16)
    y = jnp.zeros(out_ref.shape, jnp.float32)
    for k in range(n_i):
        sl = pl.ds(k * bi, bi)
        a = jax.lax.dot_general(xb, wg_ref[sl, :], (((1,), (1,)), ((), ())),
                                preferred_element_type=jnp.float32)
        b = jax.lax.dot_general(xb, wu_ref[sl, :], (((1,), (1,)), ((), ())),
                                preferred_element_type=jnp.float32)
        hh = (a * jax.lax.logistic(a) * b).astype(jnp.bfloat16)
        y = y + jax.lax.dot_general(hh, wd_ref[:, sl],
                                    (((1,), (1,)), ((), ())),
                                    preferred_element_type=jnp.float32)
    out_ref[...] = y


def _combine_body(yb_hbm, ys_hbm, p0_hbm, p1_hbm, w2_hbm, out_hbm,
                  p0v, p1v, wst, b0, b1, bs, ob, sem0, sem1, sem2,
                  *, tok_per_w):
    wid = lax.axis_index("s") * 2 + lax.axis_index("c")
    t0 = wid * tok_per_w
    ng = tok_per_w // _L
    pltpu.sync_copy(p0_hbm.at[pl.ds(t0, tok_per_w)], p0v)
    pltpu.sync_copy(p1_hbm.at[pl.ds(t0, tok_per_w)], p1v)
    pltpu.sync_copy(w2_hbm.at[pl.ds(2 * t0, 2 * tok_per_w)],
                    wst.at[pl.ds(0, 2 * tok_per_w)])

    def group(r, c):
        d0 = pltpu.async_copy(yb_hbm.at[p0v.at[pl.ds(r * _L, _L)]], b0, sem0)
        d1 = pltpu.async_copy(yb_hbm.at[p1v.at[pl.ds(r * _L, _L)]], b1, sem1)
        dsh = pltpu.async_copy(ys_hbm.at[pl.ds(t0 + r * _L, _L)], bs, sem2)
        d0.wait()
        d1.wait()
        dsh.wait()

        def row(i, c2):
            lt = r * _L + i
            wv = wst[pl.ds(2 * lt, _L)]
            w0 = wv[0]
            w1 = wv[1]

            def colf(cc, c3):
                sl = pl.ds(cc * _L, _L)
                ob[i, sl] = bs[i, sl] + w0 * b0[i, sl] + w1 * b1[i, sl]
                return c3
            lax.fori_loop(0, 64, colf, 0)
            return c2
        lax.fori_loop(0, _L, row, 0)
        pltpu.sync_copy(ob, out_hbm.at[pl.ds(t0 + r * _L, _L)])
        return c
    lax.fori_loop(0, ng, group, 0)


def kernel(x, temporal_context, tp_w, tp_b, gate_w, We_g, We_u, We_d, Ws_g, Ws_u, Ws_d):
    b, s, h = x.shape
    t = b * s
    n_exp, i_dim = We_g.shape[0], We_g.shape[1]
    n_pairs = 2 * t
    nb_routed = (n_pairs + n_exp * _BT) // _BT
    n_pad = nb_routed * _BT
    nb_pad = ((nb_routed + _L - 1) // _L) * _L

    x_flat = x.reshape(t, h)
    tc_flat = temporal_context.reshape(t, h)

    bt_r = 512
    idx2, w2 = pl.pallas_call(
        functools.partial(_router_body, n_exp=n_exp),
        grid=(t // bt_r,),
        in_specs=[
            pl.BlockSpec((bt_r, h), lambda i: (i, 0)),
            pl.BlockSpec((bt_r, h), lambda i: (i, 0)),
            pl.BlockSpec((h, h), lambda i: (0, 0)),
            pl.BlockSpec((1, h), lambda i: (0, 0)),
            pl.BlockSpec((n_exp, h), lambda i: (0, 0)),
        ],
        out_specs=[
            pl.BlockSpec((bt_r, 2), lambda i: (i, 0)),
            pl.BlockSpec((bt_r, 2), lambda i: (i, 0)),
        ],
        out_shape=[
            jax.ShapeDtypeStruct((t, 2), jnp.int32),
            jax.ShapeDtypeStruct((t, 2), jnp.float32),
        ],
    )(x_flat, tc_flat, tp_w, tp_b.reshape(1, h), gate_w)

    mesh = plsc.VectorSubcoreMesh(core_axis_name="c", subcore_axis_name="s")
    route = pl.kernel(
        functools.partial(_route_body, n_pairs=n_pairs, n_exp=n_exp,
                          n_pad=n_pad, nb_routed=nb_routed, nb_pad=nb_pad),
        out_type=[
            jax.ShapeDtypeStruct((n_pad,), jnp.int32),
            jax.ShapeDtypeStruct((t,), jnp.int32),
            jax.ShapeDtypeStruct((t,), jnp.int32),
            jax.ShapeDtypeStruct((nb_pad,), jnp.int32),
        ],
        mesh=mesh,
        scratch_types=[
            pltpu.VMEM((n_pairs,), jnp.int32),
            pltpu.VMEM((n_pairs // 128, 128), jnp.int32),
            pltpu.VMEM((n_pairs // 128, 128), jnp.int32),
            pltpu.VMEM((n_pairs,), jnp.int32),
            pltpu.VMEM((t,), jnp.int32),
            pltpu.VMEM((t,), jnp.int32),
            pltpu.VMEM((nb_pad,), jnp.int32),
            pltpu.SemaphoreType.DMA,
        ],
        compiler_params=pltpu.CompilerParams(needs_layout_passes=False),
    )
    gidx, p0, p1, bexp = route(idx2.reshape(n_pairs))

    gather = pl.kernel(
        functools.partial(_gather_body, n_groups=n_pad // _GR),
        out_type=jax.ShapeDtypeStruct((n_pad, h // 2), jnp.int32),
        mesh=mesh,
        scratch_types=[
            pltpu.VMEM((n_pad // 32,), jnp.int32),
            pltpu.VMEM((_GR, h // 2), jnp.int32),
            pltpu.VMEM((_GR, h // 2), jnp.int32),
            pltpu.SemaphoreType.DMA,
            pltpu.SemaphoreType.DMA,
        ],
        compiler_params=pltpu.CompilerParams(needs_layout_passes=False),
    )
    x16i = jax.lax.bitcast_convert_type(
        x_flat.astype(jnp.bfloat16).reshape(t, h // 2, 2), jnp.int32)
    xs_i = gather(x16i, gidx)
    xs = jax.lax.bitcast_convert_type(xs_i, jnp.bfloat16).reshape(n_pad, h)

    n_i = 2
    bi = i_dim // n_i
    wg8 = We_g.astype(jnp.bfloat16)
    wu8 = We_u.astype(jnp.bfloat16)
    wd8 = We_d.astype(jnp.bfloat16)
    yb = pl.pallas_call(
        functools.partial(_moe_body, n_i=n_i, bi=bi),
        grid_spec=pltpu.PrefetchScalarGridSpec(
            num_scalar_prefetch=1,
            grid=(nb_routed,),
            in_specs=[
                pl.BlockSpec((_BT, h), lambda bb, be: (bb, 0)),
                pl.BlockSpec((1, i_dim, h), lambda bb, be: (be[bb], 0, 0)),
                pl.BlockSpec((1, i_dim, h), lambda bb, be: (be[bb], 0, 0)),
                pl.BlockSpec((1, h, i_dim), lambda bb, be: (be[bb], 0, 0)),
            ],
            out_specs=pl.BlockSpec((_BT, h), lambda bb, be: (bb, 0)),
        ),
        out_shape=jax.ShapeDtypeStruct((n_pad, h), jnp.float32),
        compiler_params=pltpu.CompilerParams(
            dimension_semantics=("arbitrary",),
            vmem_limit_bytes=100 * 1024 * 1024,
        ),
    )(bexp, xs, wg8, wu8, wd8)

    bt_s = 512
    ys = pl.pallas_call(
        functools.partial(_shared_body, n_i=n_i, bi=bi),
        grid=(t // bt_s,),
        in_specs=[
            pl.BlockSpec((bt_s, h), lambda ti: (ti, 0)),
            pl.BlockSpec((i_dim, h), lambda ti: (0, 0)),
            pl.BlockSpec((i_dim, h), lambda ti: (0, 0)),
            pl.BlockSpec((h, i_dim), lambda ti: (0, 0)),
        ],
        out_specs=pl.BlockSpec((bt_s, h), lambda ti: (ti, 0)),
        out_shape=jax.ShapeDtypeStruct((t, h), jnp.float32),
        compiler_params=pltpu.CompilerParams(
            dimension_semantics=("arbitrary",),
        ),
    )(x_flat, Ws_g.astype(jnp.bfloat16), Ws_u.astype(jnp.bfloat16),
      Ws_d.astype(jnp.bfloat16))

    tok_per_w = t // 32
    combine = pl.kernel(
        functools.partial(_combine_body, tok_per_w=tok_per_w),
        out_type=jax.ShapeDtypeStruct((t, h), jnp.float32),
        mesh=mesh,
        scratch_types=[
            pltpu.VMEM((tok_per_w,), jnp.int32),
            pltpu.VMEM((tok_per_w,), jnp.int32),
            pltpu.VMEM((2 * tok_per_w + _L,), jnp.float32),
            pltpu.VMEM((_L, h), jnp.float32),
            pltpu.VMEM((_L, h), jnp.float32),
            pltpu.VMEM((_L, h), jnp.float32),
            pltpu.VMEM((_L, h), jnp.float32),
            pltpu.SemaphoreType.DMA,
            pltpu.SemaphoreType.DMA,
            pltpu.SemaphoreType.DMA,
        ],
        compiler_params=pltpu.CompilerParams(needs_layout_passes=False),
    )
    out = combine(yb, ys, p0, p1, w2.reshape(n_pairs))
    return out.reshape(b, s, h)


# speedup vs baseline: 1.3844x; 1.3844x over previous
import functools

import jax
import jax.numpy as jnp
from jax import lax
from jax.experimental import pallas as pl
from jax.experimental.pallas import tpu as pltpu
from jax.experimental.pallas import tpu_sc as plsc

_EPS = 1e-05
_BT = 256
_BT_LOG2 = 8
_L = 16
_GR = 32


def _router_body(x_ref, tc_ref, tpw_ref, tpb_ref, gw_ref, idx_ref, w_ref, *, n_exp):
    x_r = x_ref[...] + jax.lax.dot_general(
        tc_ref[...], tpw_ref[...], (((1,), (1,)), ((), ())),
        preferred_element_type=jnp.float32) + tpb_ref[...]
    logits = jax.lax.dot_general(x_r, gw_ref[...], (((1,), (1,)), ((), ())),
                                 preferred_element_type=jnp.float32)
    m = jnp.max(logits, axis=1, keepdims=True)
    p = jnp.exp(logits - m)
    probs = p / jnp.sum(p, axis=1, keepdims=True)
    it = jax.lax.broadcasted_iota(jnp.int32, probs.shape, 1)
    m0 = jnp.max(probs, axis=1, keepdims=True)
    i0 = jnp.min(jnp.where(probs == m0, it, n_exp), axis=1, keepdims=True)
    pm = jnp.where(it == i0, -1.0, probs)
    m1 = jnp.max(pm, axis=1, keepdims=True)
    i1 = jnp.min(jnp.where(pm == m1, it, n_exp), axis=1, keepdims=True)
    denom = m0 + m1 + _EPS
    w0 = m0 / denom
    w1 = m1 / denom
    it2 = jax.lax.broadcasted_iota(jnp.int32, idx_ref.shape, 1)
    idx_ref[...] = jnp.where(it2 == 0, i0, i1)
    w_ref[...] = jnp.where(it2 == 0, w0, w1)


def _route_body(ebuf_hbm, gidx_hbm, p0_hbm, p1_hbm, bexp_hbm,
                ebuf, posbuf, tokbuf, stage, p0s, p1s, bexpv, sem,
                *, n_pairs, n_exp, n_pad, nb_routed, nb_pad):
    wid = lax.axis_index("s") * 2 + lax.axis_index("c")

    @pl.when(wid == 0)
    def _work():
        lanes = lax.iota(jnp.int32, _L)
        nv = n_pairs // _L

        pltpu.sync_copy(ebuf_hbm, ebuf)

        def hist_step(i, tot):
            v = ebuf[pl.ds(i * _L, _L)]
            for e in range(n_exp):
                cnt = jnp.sum(jnp.where(v == e, 1, 0))
                tot = jnp.where(lanes == e, tot + cnt, tot)
            return tot
        tot = lax.fori_loop(0, nv, hist_step, jnp.zeros((_L,), jnp.int32))

        inc = jax.lax.shift_left(
            jax.lax.shift_right_logical(tot + (_BT - 1), _BT_LOG2), _BT_LOG2)
        pstart = plsc.cumsum(inc) - inc

        pf = [jnp.sum(jnp.where(lanes == f, pstart, 0)) for f in range(n_exp)]
        nbv = nb_pad // _L
        for i in range(nbv):
            bv = lanes + i * _L
            start = bv * _BT
            cnt = jnp.zeros((_L,), jnp.int32)
            for f in range(n_exp):
                cnt = cnt + jnp.where(start >= pf[f], 1, 0)
            ev = jnp.minimum(jnp.maximum(cnt - 1, 0), n_exp - 1)
            bexpv[pl.ds(i * _L, _L)] = ev
        pltpu.sync_copy(bexpv, bexp_hbm)

        def zfill(i, c):
            stage[pl.ds(i * _L, _L)] = jnp.zeros((_L,), jnp.int32)
            return c
        lax.fori_loop(0, n_pairs // _L, zfill, 0)
        pltpu.sync_copy(stage, gidx_hbm.at[pl.ds(0, n_pairs)])
        rest = n_pad - n_pairs
        pltpu.sync_copy(stage.at[pl.ds(0, rest)],
                        gidx_hbm.at[pl.ds(n_pairs, rest)])

        def pos_step(i, cur):
            v = ebuf[pl.ds(i * _L, _L)]
            pos = jnp.zeros((_L,), jnp.int32)
            for e in range(n_exp):
                msk = v == e
                mi = jnp.where(msk, 1, 0)
                pre = plsc.cumsum(mi) - mi
                cnt = jnp.sum(mi)
                base = jnp.sum(jnp.where(lanes == e, cur, 0))
                pos = jnp.where(msk, base + pre, pos)
                cur = jnp.where(lanes == e, cur + cnt, cur)
            row = jax.lax.shift_right_logical(i, 3)
            col = jax.lax.shift_left(jnp.bitwise_and(i, 7), 4)
            posbuf[row, pl.ds(col, _L)] = pos
            tokbuf[row, pl.ds(col, _L)] = jax.lax.shift_right_logical(
                i * _L + lanes, 1)
            return cur
        lax.fori_loop(0, nv, pos_step, pstart)

        nrow = n_pairs // 128

        def sc_out(r8, c):
            for r2 in range(8):
                r = r8 * 8 + r2
                pltpu.async_copy(tokbuf.at[r], gidx_hbm.at[posbuf.at[r]], sem)
            for r2 in range(8):
                r = r8 * 8 + r2
                pltpu.make_async_copy(
                    tokbuf.at[r], gidx_hbm.at[posbuf.at[r]], sem).wait()
            return c
        lax.fori_loop(0, nrow // 8, sc_out, 0)

        def dei(g, c):
            slot = jax.lax.shift_left(g * _L + lanes, 1)
            r0 = jax.lax.shift_right_logical(slot, 7)
            c0 = jnp.bitwise_and(slot, 127)
            p0s[pl.ds(g * _L, _L)] = plsc.load_gather(posbuf, [r0, c0])
            slot1 = slot + 1
            r1 = jax.lax.shift_right_logical(slot1, 7)
            c1 = jnp.bitwise_and(slot1, 127)
            p1s[pl.ds(g * _L, _L)] = plsc.load_gather(posbuf, [r1, c1])
            return c
        lax.fori_loop(0, (n_pairs // 2) // _L, dei, 0)
        pltpu.sync_copy(p0s, p0_hbm)
        pltpu.sync_copy(p1s, p1_hbm)


def _gather_body(x_hbm, gidx_hbm, xs_hbm, gxbuf, rows0, rows1, semA, semB,
                 *, n_groups):
    wid = lax.axis_index("s") * 2 + lax.axis_index("c")
    per_w = n_groups // 32
    g0 = wid * per_w
    pltpu.sync_copy(gidx_hbm.at[pl.ds(g0 * _GR, per_w * _GR)], gxbuf)

    def idx(g):
        return gxbuf.at[pl.ds(g * _GR, _GR)]

    pltpu.async_copy(x_hbm.at[idx(0)], rows0, semA)

    def step(k, c):
        ge = 2 * k
        go = 2 * k + 1
        pltpu.async_copy(x_hbm.at[idx(go)], rows1, semB)
        pltpu.make_async_copy(x_hbm.at[idx(ge)], rows0, semA).wait()
        pltpu.sync_copy(rows0, xs_hbm.at[pl.ds((g0 + ge) * _GR, _GR)])

        @pl.when(k < per_w // 2 - 1)
        def _prefetch():
            pltpu.async_copy(x_hbm.at[idx(ge + 2)], rows0, semA)

        pltpu.make_async_copy(x_hbm.at[idx(go)], rows1, semB).wait()
        pltpu.sync_copy(rows1, xs_hbm.at[pl.ds((g0 + go) * _GR, _GR)])
        return c
    lax.fori_loop(0, per_w // 2, step, 0)


def _moe_body(be_ref, xs_ref, wg_ref, wu_ref, wd_ref, out_ref, *, n_i, bi):
    xb = xs_ref[...].astype(jnp.bfloat16)
    y = jnp.zeros(out_ref.shape, jnp.float32)
    for k in range(n_i):
        sl = pl.ds(k * bi, bi)
        a = jax.lax.dot_general(xb, wg_ref[0, sl, :], (((1,), (1,)), ((), ())),
                                preferred_element_type=jnp.float32)
        b = jax.lax.dot_general(xb, wu_ref[0, sl, :], (((1,), (1,)), ((), ())),
                                preferred_element_type=jnp.float32)
        hh = (a * jax.lax.logistic(a) * b).astype(jnp.bfloat16)
        y = y + jax.lax.dot_general(hh, wd_ref[0, :, sl],
                                    (((1,), (1,)), ((), ())),
                                    preferred_element_type=jnp.float32)
    out_ref[...] = y


def _shared_body(x_ref, wg_ref, wu_ref, wd_ref, out_ref, *, n_i, bi):
    xb = x_ref[...].astype(jnp.bfloat16)
    y = jnp.zeros(out_ref.shape, jnp.float32)
    for k in range(n_i):
        sl = pl.ds(k * bi, bi)
        a = jax.lax.dot_general(xb, wg_ref[sl, :], (((1,), (1,)), ((), ())),
                                preferred_element_type=jnp.float32)
        b = jax.lax.dot_general(xb, wu_ref[sl, :], (((1,), (1,)), ((), ())),
                                preferred_element_type=jnp.float32)
        hh = (a * jax.lax.logistic(a) * b).astype(jnp.bfloat16)
        y = y + jax.lax.dot_general(hh, wd_ref[:, sl],
                                    (((1,), (1,)), ((), ())),
                                    preferred_element_type=jnp.float32)
    out_ref[...] = y


def _combine_body(yb_hbm, ys_hbm, p0_hbm, p1_hbm, w2_hbm, out_hbm,
                  p0v, p1v, wst, b0a, b1a, bsa, b0b, b1b, bsb, ob,
                  sem0, sem1, sem2, sem3, sem4, sem5,
                  *, tok_per_w, h):
    wid = lax.axis_index("s") * 2 + lax.axis_index("c")
    t0 = wid * tok_per_w
    ng = tok_per_w // _L
    nc = h // _L
    pltpu.sync_copy(p0_hbm.at[pl.ds(t0, tok_per_w)], p0v)
    pltpu.sync_copy(p1_hbm.at[pl.ds(t0, tok_per_w)], p1v)
    pltpu.sync_copy(w2_hbm.at[pl.ds(2 * t0, 2 * tok_per_w)],
                    wst.at[pl.ds(0, 2 * tok_per_w)])

    def fire(r, b0, b1, bs, s0, s1, s2):
        pltpu.async_copy(yb_hbm.at[p0v.at[pl.ds(r * _L, _L)]], b0, s0)
        pltpu.async_copy(yb_hbm.at[p1v.at[pl.ds(r * _L, _L)]], b1, s1)
        pltpu.async_copy(ys_hbm.at[pl.ds(t0 + r * _L, _L)], bs, s2)

    def drain(r, b0, b1, bs, s0, s1, s2):
        pltpu.make_async_copy(
            yb_hbm.at[p0v.at[pl.ds(r * _L, _L)]], b0, s0).wait()
        pltpu.make_async_copy(
            yb_hbm.at[p1v.at[pl.ds(r * _L, _L)]], b1, s1).wait()
        pltpu.make_async_copy(
            ys_hbm.at[pl.ds(t0 + r * _L, _L)], bs, s2).wait()

    def compute(r, b0, b1, bs):
        def row(i, c2):
            lt = r * _L + i
            wv = wst[pl.ds(2 * lt, _L)]
            w0 = wv[0]
            w1 = wv[1]
            for cc in range(nc):
                sl = pl.ds(cc * _L, _L)
                ob[i, sl] = bs[i, sl] + w0 * b0[i, sl] + w1 * b1[i, sl]
            return c2
        lax.fori_loop(0, _L, row, 0)
        pltpu.sync_copy(ob, out_hbm.at[pl.ds(t0 + r * _L, _L)])

    fire(0, b0a, b1a, bsa, sem0, sem1, sem2)

    def step(k, c):
        ge = 2 * k
        go = 2 * k + 1
        fire(go, b0b, b1b, bsb, sem3, sem4, sem5)
        drain(ge, b0a, b1a, bsa, sem0, sem1, sem2)
        compute(ge, b0a, b1a, bsa)

        @pl.when(k < ng // 2 - 1)
        def _pref():
            fire(ge + 2, b0a, b1a, bsa, sem0, sem1, sem2)

        drain(go, b0b, b1b, bsb, sem3, sem4, sem5)
        compute(go, b0b, b1b, bsb)
        return c
    lax.fori_loop(0, ng // 2, step, 0)


def kernel(x, temporal_context, tp_w, tp_b, gate_w, We_g, We_u, We_d, Ws_g, Ws_u, Ws_d):
    b, s, h = x.shape
    t = b * s
    n_exp, i_dim = We_g.shape[0], We_g.shape[1]
    n_pairs = 2 * t
    nb_routed = (n_pairs + n_exp * _BT) // _BT
    n_pad = nb_routed * _BT
    nb_pad = ((nb_routed + _L - 1) // _L) * _L

    x_flat = x.reshape(t, h)
    tc_flat = temporal_context.reshape(t, h)

    bt_r = 512
    idx2, w2 = pl.pallas_call(
        functools.partial(_router_body, n_exp=n_exp),
        grid=(t // bt_r,),
        in_specs=[
            pl.BlockSpec((bt_r, h), lambda i: (i, 0)),
            pl.BlockSpec((bt_r, h), lambda i: (i, 0)),
            pl.BlockSpec((h, h), lambda i: (0, 0)),
            pl.BlockSpec((1, h), lambda i: (0, 0)),
            pl.BlockSpec((n_exp, h), lambda i: (0, 0)),
        ],
        out_specs=[
            pl.BlockSpec((bt_r, 2), lambda i: (i, 0)),
            pl.BlockSpec((bt_r, 2), lambda i: (i, 0)),
        ],
        out_shape=[
            jax.ShapeDtypeStruct((t, 2), jnp.int32),
            jax.ShapeDtypeStruct((t, 2), jnp.float32),
        ],
    )(x_flat, tc_flat, tp_w, tp_b.reshape(1, h), gate_w)

    mesh = plsc.VectorSubcoreMesh(core_axis_name="c", subcore_axis_name="s")
    route = pl.kernel(
        functools.partial(_route_body, n_pairs=n_pairs, n_exp=n_exp,
                          n_pad=n_pad, nb_routed=nb_routed, nb_pad=nb_pad),
        out_type=[
            jax.ShapeDtypeStruct((n_pad,), jnp.int32),
            jax.ShapeDtypeStruct((t,), jnp.int32),
            jax.ShapeDtypeStruct((t,), jnp.int32),
            jax.ShapeDtypeStruct((nb_pad,), jnp.int32),
        ],
        mesh=mesh,
        scratch_types=[
            pltpu.VMEM((n_pairs,), jnp.int32),
            pltpu.VMEM((n_pairs // 128, 128), jnp.int32),
            pltpu.VMEM((n_pairs // 128, 128), jnp.int32),
            pltpu.VMEM((n_pairs,), jnp.int32),
            pltpu.VMEM((t,), jnp.int32),
            pltpu.VMEM((t,), jnp.int32),
            pltpu.VMEM((nb_pad,), jnp.int32),
            pltpu.SemaphoreType.DMA,
        ],
        compiler_params=pltpu.CompilerParams(needs_layout_passes=False),
    )
    gidx, p0, p1, bexp = route(idx2.reshape(n_pairs))

    gather = pl.kernel(
        functools.partial(_gather_body, n_groups=n_pad // _GR),
        out_type=jax.ShapeDtypeStruct((n_pad, h), jnp.float32),
        mesh=mesh,
        scratch_types=[
            pltpu.VMEM((n_pad // 32,), jnp.int32),
            pltpu.VMEM((_GR, h), jnp.float32),
            pltpu.VMEM((_GR, h), jnp.float32),
            pltpu.SemaphoreType.DMA,
            pltpu.SemaphoreType.DMA,
        ],
        compiler_params=pltpu.CompilerParams(needs_layout_passes=False),
    )
    xs = gather(x_flat, gidx)

    n_i = 4
    bi = i_dim // n_i
    wg8 = We_g.astype(jnp.bfloat16)
    wu8 = We_u.astype(jnp.bfloat16)
    wd8 = We_d.astype(jnp.bfloat16)
    yb = pl.pallas_call(
        functools.partial(_moe_body, n_i=n_i, bi=bi),
        grid_spec=pltpu.PrefetchScalarGridSpec(
            num_scalar_prefetch=1,
            grid=(nb_routed,),
            in_specs=[
                pl.BlockSpec((_BT, h), lambda bb, be: (bb, 0)),
                pl.BlockSpec((1, i_dim, h), lambda bb, be: (be[bb], 0, 0)),
                pl.BlockSpec((1, i_dim, h), lambda bb, be: (be[bb], 0, 0)),
                pl.BlockSpec((1, h, i_dim), lambda bb, be: (be[bb], 0, 0)),
            ],
            out_specs=pl.BlockSpec((_BT, h), lambda bb, be: (bb, 0)),
        ),
        out_shape=jax.ShapeDtypeStruct((n_pad, h), jnp.float32),
        compiler_params=pltpu.CompilerParams(
            dimension_semantics=("arbitrary",),
            vmem_limit_bytes=100 * 1024 * 1024,
        ),
    )(bexp, xs, wg8, wu8, wd8)

    bt_s = 512
    ys = pl.pallas_call(
        functools.partial(_shared_body, n_i=n_i, bi=bi),
        grid=(t // bt_s,),
        in_specs=[
            pl.BlockSpec((bt_s, h), lambda ti: (ti, 0)),
            pl.BlockSpec((i_dim, h), lambda ti: (0, 0)),
            pl.BlockSpec((i_dim, h), lambda ti: (0, 0)),
            pl.BlockSpec((h, i_dim), lambda ti: (0, 0)),
        ],
        out_specs=pl.BlockSpec((bt_s, h), lambda ti: (ti, 0)),
        out_shape=jax.ShapeDtypeStruct((t, h), jnp.float32),
        compiler_params=pltpu.CompilerParams(
            dimension_semantics=("arbitrary",),
        ),
    )(x_flat, Ws_g.astype(jnp.bfloat16), Ws_u.astype(jnp.bfloat16),
      Ws_d.astype(jnp.bfloat16))

    tok_per_w = t // 32
    combine = pl.kernel(
        functools.partial(_combine_body, tok_per_w=tok_per_w, h=h),
        out_type=jax.ShapeDtypeStruct((t, h), jnp.float32),
        mesh=mesh,
        scratch_types=[
            pltpu.VMEM((tok_per_w,), jnp.int32),
            pltpu.VMEM((tok_per_w,), jnp.int32),
            pltpu.VMEM((2 * tok_per_w + _L,), jnp.float32),
            pltpu.VMEM((_L, h), jnp.float32),
            pltpu.VMEM((_L, h), jnp.float32),
            pltpu.VMEM((_L, h), jnp.float32),
            pltpu.VMEM((_L, h), jnp.float32),
            pltpu.VMEM((_L, h), jnp.float32),
            pltpu.VMEM((_L, h), jnp.float32),
            pltpu.VMEM((_L, h), jnp.float32),
            pltpu.SemaphoreType.DMA,
            pltpu.SemaphoreType.DMA,
            pltpu.SemaphoreType.DMA,
            pltpu.SemaphoreType.DMA,
            pltpu.SemaphoreType.DMA,
            pltpu.SemaphoreType.DMA,
        ],
        compiler_params=pltpu.CompilerParams(needs_layout_passes=False),
    )
    out = combine(yb, ys, p0, p1, w2.reshape(n_pairs))
    return out.reshape(b, s, h)
